# baseline (speedup 1.0000x reference)
import jax
import jax.numpy as jnp
from jax.experimental import pallas as pl
from jax.experimental.pallas import tpu as pltpu

_IN = 64
_OUT = 2
_PACK = 64
_LANES_IN = _PACK * _IN
_LANES_OUT = _PACK * _OUT

_TILE = 256
_MIN_SPLIT = 32


def _linear_body(x_ref, w_ref, b_ref, o_ref):
    acc = jax.lax.dot_general(
        x_ref[...], w_ref[...],
        dimension_numbers=(((1,), (0,)), ((), ())),
        preferred_element_type=jnp.float32,
    )
    o_ref[...] = (acc + b_ref[...]).astype(o_ref.dtype)


def kernel(x, weight, bias):
    B = x.shape[0]
    dtype = x.dtype

    B_work = ((B + _PACK - 1) // _PACK) * _PACK
    if B_work != B:
        x = jnp.pad(x, ((0, B_work - B), (0, 0)))

    P = B_work // _PACK
    x_packed = x.reshape(P, _LANES_IN)

    wt = weight.T.astype(dtype)
    w_big = jnp.kron(jnp.eye(_PACK, dtype=dtype), wt)
    b_big = jnp.tile(bias.astype(dtype), _PACK).reshape(1, _LANES_OUT)

    if P >= 2 * _TILE:
        tile = _TILE
    elif P >= _MIN_SPLIT:
        tile = ((pl.cdiv(P, 2) + 7) // 8) * 8
    else:
        tile = P
    grid = (pl.cdiv(P, tile),)

    out_packed = pl.pallas_call(
        _linear_body,
        out_shape=jax.ShapeDtypeStruct((P, _LANES_OUT), dtype),
        grid=grid,
        in_specs=[
            pl.BlockSpec((tile, _LANES_IN), lambda i: (i, 0)),
            pl.BlockSpec((_LANES_IN, _LANES_OUT), lambda i: (0, 0)),
            pl.BlockSpec((1, _LANES_OUT), lambda i: (0, 0)),
        ],
        out_specs=pl.BlockSpec((tile, _LANES_OUT), lambda i: (i, 0)),
        compiler_params=pltpu.CompilerParams(
            dimension_semantics=("parallel",),
        ),
    )(x_packed, w_big, b_big)

    out = out_packed.reshape(B_work, _OUT)
    return out if B_work == B else out[:B]


# speedup vs baseline: 1.7509x; 1.7509x over previous
import jax
import jax.numpy as jnp
from jax.experimental import pallas as pl
from jax.experimental.pallas import tpu as pltpu

_IN = 64
_OUT = 2

_TILE_B = 16384
_MIN_SPLIT = 256


def _linear_body(x_ref, w_ref, b_ref, o_ref):
    acc = jax.lax.dot_general(
        x_ref[...], w_ref[...],
        dimension_numbers=(((1,), (1,)), ((), ())),
        preferred_element_type=jnp.float32,
    )
    o_ref[...] = (acc + b_ref[...]).astype(o_ref.dtype)


def kernel(x, weight, bias):
    B = x.shape[0]
    dtype = x.dtype

    if B >= 2 * _TILE_B:
        tile = _TILE_B
    elif B >= _MIN_SPLIT:
        tile = ((pl.cdiv(B, 2) + 7) // 8) * 8
    else:
        tile = B
    grid = (pl.cdiv(B, tile),)

    b2d = bias.astype(dtype).reshape(1, _OUT)

    return pl.pallas_call(
        _linear_body,
        out_shape=jax.ShapeDtypeStruct((B, _OUT), dtype),
        grid=grid,
        in_specs=[
            pl.BlockSpec((tile, _IN), lambda i: (i, 0)),
            pl.BlockSpec((_OUT, _IN), lambda i: (0, 0)),
            pl.BlockSpec((1, _OUT), lambda i: (0, 0)),
        ],
        out_specs=pl.BlockSpec((tile, _OUT), lambda i: (i, 0)),
        compiler_params=pltpu.CompilerParams(
            dimension_semantics=("parallel",),
        ),
    )(x, weight.astype(dtype), b2d)
